# baseline (speedup 1.0000x reference)
import jax
import jax.numpy as jnp
import numpy as np
from jax import lax
from jax.experimental import pallas as pl

_N = 2048
_K = 64
_R = 16
_C = _N // _R

_HI = jax.lax.Precision.HIGHEST


def _dot_tn(a, b, precision=None):
    return lax.dot_general(a, b, (((0,), (0,)), ((), ())), precision=precision)


def _dot_nn(a, b, precision=None):
    return lax.dot_general(a, b, (((1,), (0,)), ((), ())), precision=precision)


def _dot_nt(a, b, precision=None):
    return lax.dot_general(a, b, (((1,), (1,)), ((), ())), precision=precision)


def _body(src_ref, w1_ref, b1_ref, w2_ref, b2_ref, ws_ref, out_ref):
    def score(a):
        h1 = jnp.maximum(_dot_tn(w1_ref[...], a) + b1_ref[...], 0.0)
        h2 = jnp.maximum(_dot_tn(w2_ref[...], h1) + b2_ref[...], 0.0)
        return _dot_tn(ws_ref[...], h2)

    s_row = score(src_ref[0]) + score(src_ref[1])

    s2d = jnp.concatenate(
        [s_row[:, i * _C:(i + 1) * _C] for i in range(_R)], axis=0)

    ii = lax.bitcast_convert_type(s2d, jnp.int32)
    imin = jnp.int32(-2147483648)
    mono = jnp.where(ii >= 0, ii, imin - ii)

    t = jnp.full((1, 1), imin, jnp.int32)
    for i in range(32):
        cand = t + jnp.int32(np.int32(np.uint32(1 << (31 - i))))
        cnt = jnp.sum((mono >= cand).astype(jnp.int32), axis=(0, 1),
                      keepdims=True)
        t = jnp.where(cnt >= _K, cand, t)

    gt = (mono > t).astype(jnp.float32)
    eq = (mono == t).astype(jnp.float32)
    c_gt = jnp.sum(gt)

    r_io = lax.broadcasted_iota(jnp.int32, (_C, _C), 0)
    c_io = lax.broadcasted_iota(jnp.int32, (_C, _C), 1)
    upper = (r_io <= c_io).astype(jnp.float32)
    r16 = lax.broadcasted_iota(jnp.int32, (_R, _R), 0)
    c16 = lax.broadcasted_iota(jnp.int32, (_R, _R), 1)
    lower16 = (c16 < r16).astype(jnp.float32)

    def excl_prefix(m):
        rowcum = _dot_nn(m, upper, precision=_HI)
        prior = _dot_nn(lower16, rowcum[:, _C - 1:_C], precision=_HI)
        return rowcum + prior - m

    sel_eq = eq * (excl_prefix(eq) < (_K - c_gt)).astype(jnp.float32)
    sel = gt + sel_eq
    cpos = excl_prefix(sel)

    sel_row = jnp.concatenate(
        [sel[i:i + 1, :] for i in range(_R)], axis=1)
    cpos_row = jnp.concatenate(
        [cpos[i:i + 1, :] for i in range(_R)], axis=1).astype(jnp.int32)

    kio = lax.broadcasted_iota(jnp.int32, (_K, _N), 0)
    p = jnp.where((kio == cpos_row) & (sel_row > 0.5), 1.0, 0.0)

    cs_row = _dot_nt(s_row, p, precision=_HI)
    cs_col = jnp.transpose(cs_row)

    a_io = lax.broadcasted_iota(jnp.int32, (_K, _K), 0)
    b_io = lax.broadcasted_iota(jnp.int32, (_K, _K), 1)
    before = (cs_row > cs_col) | ((cs_row == cs_col) & (b_io < a_io))
    r_col = jnp.sum(before.astype(jnp.float32), axis=1, keepdims=True)
    r_row = jnp.transpose(r_col).astype(jnp.int32)
    ro = (a_io == r_row).astype(jnp.float32)

    for b in range(2):
        pts = _dot_nt(p, src_ref[b], precision=_HI)
        out_ref[b] = _dot_nn(ro, pts, precision=_HI)


def kernel(src_pts, tgt_pts, W1, b1, W2, b2, Ws, bs):
    del tgt_pts, bs
    call = pl.pallas_call(
        _body,
        out_shape=jax.ShapeDtypeStruct((2, _K, 6), jnp.float32),
    )
    return call(src_pts, W1, b1[:, None], W2, b2[:, None], Ws)


# speedup vs baseline: 1.0850x; 1.0850x over previous
import jax
import jax.numpy as jnp
from jax import lax
from jax.experimental import pallas as pl

_N = 2048
_K = 64
_R = 16
_C = _N // _R

_HI = jax.lax.Precision.HIGHEST


def _dot_tn(a, b, precision=None):
    return lax.dot_general(a, b, (((0,), (0,)), ((), ())), precision=precision)


def _dot_nn(a, b, precision=None):
    return lax.dot_general(a, b, (((1,), (0,)), ((), ())), precision=precision)


def _dot_nt(a, b, precision=None):
    return lax.dot_general(a, b, (((1,), (1,)), ((), ())), precision=precision)


def _monotonic(x):
    ii = lax.bitcast_convert_type(x, jnp.int32)
    imin = jnp.int32(-2147483648)
    return jnp.where(ii >= 0, ii, imin - ii)


def _body(src_ref, w1_ref, b1_ref, w2_ref, b2_ref, ws_ref, out_ref):
    imin = jnp.int32(-2147483648)

    def score(a):
        h1 = jnp.maximum(_dot_tn(w1_ref[...], a) + b1_ref[...], 0.0)
        h2 = jnp.maximum(_dot_tn(w2_ref[...], h1) + b2_ref[...], 0.0)
        return _dot_tn(ws_ref[...], h2)

    s_row = score(src_ref[0]) + score(src_ref[1])

    s2d = jnp.concatenate(
        [s_row[:, i * _C:(i + 1) * _C] for i in range(_R)], axis=0)
    mono = _monotonic(s2d)
    u_row = _monotonic(s_row) ^ imin

    kio = lax.broadcasted_iota(jnp.int32, (_K, _N), 0)
    a_io = lax.broadcasted_iota(jnp.int32, (_K, _K), 0)
    b_io = lax.broadcasted_iota(jnp.int32, (_K, _K), 1)
    uge64 = (b_io >= a_io).astype(jnp.float32)
    jcol = lax.broadcasted_iota(jnp.int32, (_K, 1), 0)
    ones_row = jnp.ones((1, _N), jnp.float32)

    active = jnp.ones((1, _N), jnp.float32)
    above = jnp.zeros((1, 1), jnp.float32)
    u_t = jnp.zeros((1, 1), jnp.int32)
    for shift, bits in ((26, 6), (20, 6), (14, 6), (8, 6), (2, 6), (0, 2)):
        nb = 1 << bits
        digit = lax.shift_right_logical(u_row, shift) & jnp.int32(nb - 1)
        hot = ((kio[:nb] == digit) & (active > 0.5)).astype(jnp.float32)
        hist = _dot_nt(hot, ones_row)
        t_cum = above + _dot_nn(uge64[:nb, :nb], hist)
        pick = t_cum >= jnp.float32(_K)
        jstar = jnp.max(jnp.where(pick, jcol[:nb], -1), axis=(0, 1),
                        keepdims=True)
        at_j = (jcol[:nb] == jstar)
        above = jnp.sum(jnp.where(at_j, t_cum - hist, 0.0), axis=(0, 1),
                        keepdims=True)
        active = active * (digit == jstar).astype(jnp.float32)
        u_t = u_t | lax.shift_left(jstar, shift)

    t = (u_t ^ imin)

    gt = (mono > t).astype(jnp.float32)
    eq = (mono == t).astype(jnp.float32)
    c_gt = jnp.sum(gt)

    r_io = lax.broadcasted_iota(jnp.int32, (_C, _C), 0)
    c_io = lax.broadcasted_iota(jnp.int32, (_C, _C), 1)
    upper = (r_io <= c_io).astype(jnp.float32)
    r16 = lax.broadcasted_iota(jnp.int32, (_R, _R), 0)
    c16 = lax.broadcasted_iota(jnp.int32, (_R, _R), 1)
    lower16 = (c16 < r16).astype(jnp.float32)

    def excl_prefix(m):
        rowcum = _dot_nn(m, upper)
        prior = _dot_nn(lower16, rowcum[:, _C - 1:_C])
        return rowcum + prior - m

    sel_eq = eq * (excl_prefix(eq) < (_K - c_gt)).astype(jnp.float32)
    sel = gt + sel_eq
    cpos = excl_prefix(sel)

    sel_row = jnp.concatenate(
        [sel[i:i + 1, :] for i in range(_R)], axis=1)
    cpos_row = jnp.concatenate(
        [cpos[i:i + 1, :] for i in range(_R)], axis=1).astype(jnp.int32)

    p = jnp.where((kio == cpos_row) & (sel_row > 0.5), 1.0, 0.0)

    cs_row = _dot_nt(s_row, p, precision=_HI)
    cs_col = jnp.transpose(cs_row)

    before = (cs_row > cs_col) | ((cs_row == cs_col) & (b_io < a_io))
    r_col = jnp.sum(before.astype(jnp.float32), axis=1, keepdims=True)
    r_row = jnp.transpose(r_col).astype(jnp.int32)
    ro = (a_io == r_row).astype(jnp.float32)
    g = _dot_nn(ro, p)

    for b in range(2):
        x = src_ref[b]
        hi = x.astype(jnp.bfloat16).astype(jnp.float32)
        mid = (x - hi).astype(jnp.bfloat16).astype(jnp.float32)
        lo = x - hi - mid
        out_ref[b] = (_dot_nt(g, hi) + _dot_nt(g, mid)) + _dot_nt(g, lo)


def kernel(src_pts, tgt_pts, W1, b1, W2, b2, Ws, bs):
    del tgt_pts, bs
    call = pl.pallas_call(
        _body,
        out_shape=jax.ShapeDtypeStruct((2, _K, 6), jnp.float32),
    )
    return call(src_pts, W1, b1[:, None], W2, b2[:, None], Ws)


# speedup vs baseline: 1.1614x; 1.0705x over previous
import jax
import jax.numpy as jnp
from jax import lax
from jax.experimental import pallas as pl

_N = 2048
_K = 64
_R = 16
_C = _N // _R

_HI = jax.lax.Precision.HIGHEST


def _dot_tn(a, b, precision=None):
    return lax.dot_general(a, b, (((0,), (0,)), ((), ())), precision=precision)


def _dot_nn(a, b, precision=None):
    return lax.dot_general(a, b, (((1,), (0,)), ((), ())), precision=precision)


def _dot_nt(a, b, precision=None):
    return lax.dot_general(a, b, (((1,), (1,)), ((), ())), precision=precision)


def _monotonic(x):
    ii = lax.bitcast_convert_type(x, jnp.int32)
    imin = jnp.int32(-2147483648)
    return jnp.where(ii >= 0, ii, imin - ii)


def _body(src_ref, w1_ref, b1_ref, w2_ref, b2_ref, ws_ref, out_ref):
    imin = jnp.int32(-2147483648)

    def score(a):
        h1 = jnp.maximum(_dot_tn(w1_ref[...], a) + b1_ref[...], 0.0)
        h2 = jnp.maximum(_dot_tn(w2_ref[...], h1) + b2_ref[...], 0.0)
        return _dot_tn(ws_ref[...], h2)

    s_row = score(src_ref[0]) + score(src_ref[1])

    s2d = jnp.concatenate(
        [s_row[:, i * _C:(i + 1) * _C] for i in range(_R)], axis=0)
    mono = _monotonic(s2d)
    u_row = _monotonic(s_row) ^ imin

    kio = lax.broadcasted_iota(jnp.int32, (_K, _N), 0)
    a_io = lax.broadcasted_iota(jnp.int32, (_K, _K), 0)
    b_io = lax.broadcasted_iota(jnp.int32, (_K, _K), 1)
    uge64 = (b_io >= a_io).astype(jnp.float32)
    jcol = lax.broadcasted_iota(jnp.int32, (_K, 1), 0)
    ones_row = jnp.ones((1, _N), jnp.float32)

    active = jnp.ones((1, _N), jnp.float32)
    above = jnp.zeros((1, 1), jnp.float32)
    u_t = jnp.zeros((1, 1), jnp.int32)
    for shift, bits in ((26, 6), (20, 6), (14, 6), (8, 6), (2, 6), (0, 2)):
        nb = 1 << bits
        digit = lax.shift_right_logical(u_row, shift) & jnp.int32(nb - 1)
        digit = jnp.where(active > 0.5, digit, jnp.int32(nb))
        hot = (kio[:nb] == digit).astype(jnp.float32)
        hist = _dot_nt(hot, ones_row)
        t_cum = above + _dot_nn(uge64[:nb, :nb], hist)
        pick = t_cum >= jnp.float32(_K)
        jstar = jnp.max(jnp.where(pick, jcol[:nb], -1), axis=(0, 1),
                        keepdims=True)
        at_j = (jcol[:nb] == jstar)
        above = jnp.sum(jnp.where(at_j, t_cum - hist, 0.0), axis=(0, 1),
                        keepdims=True)
        active = (digit == jstar).astype(jnp.float32)
        u_t = u_t | lax.shift_left(jstar, shift)

    t = (u_t ^ imin)

    gt = (mono > t).astype(jnp.float32)
    eq = (mono == t).astype(jnp.float32)
    c_gt = jnp.sum(gt)

    r_io = lax.broadcasted_iota(jnp.int32, (_C, _C), 0)
    c_io = lax.broadcasted_iota(jnp.int32, (_C, _C), 1)
    upper = (r_io <= c_io).astype(jnp.float32)
    r16 = lax.broadcasted_iota(jnp.int32, (_R, _R), 0)
    c16 = lax.broadcasted_iota(jnp.int32, (_R, _R), 1)
    lower16 = (c16 < r16).astype(jnp.float32)

    def excl_prefix(m):
        rowcum = _dot_nn(m, upper)
        prior = _dot_nn(lower16, rowcum[:, _C - 1:_C])
        return rowcum + prior - m

    sel_eq = eq * (excl_prefix(eq) < (_K - c_gt)).astype(jnp.float32)
    sel = gt + sel_eq
    cpos = excl_prefix(sel)

    sel_row = jnp.concatenate(
        [sel[i:i + 1, :] for i in range(_R)], axis=1)
    cpos_row = jnp.concatenate(
        [cpos[i:i + 1, :] for i in range(_R)], axis=1).astype(jnp.int32)

    p = jnp.where((kio == cpos_row) & (sel_row > 0.5), 1.0, 0.0)

    s_hi = s_row.astype(jnp.bfloat16).astype(jnp.float32)
    s_mid = (s_row - s_hi).astype(jnp.bfloat16).astype(jnp.float32)
    s_lo = s_row - s_hi - s_mid
    cs_row = (_dot_nt(s_hi, p) + _dot_nt(s_mid, p)) + _dot_nt(s_lo, p)
    cs_col = jnp.transpose(cs_row)

    before = (cs_row > cs_col) | ((cs_row == cs_col) & (b_io < a_io))
    r_col = jnp.sum(before.astype(jnp.float32), axis=1, keepdims=True)
    r_row = jnp.transpose(r_col).astype(jnp.int32)
    ro = (a_io == r_row).astype(jnp.float32)
    g = _dot_nn(ro, p)

    for b in range(2):
        x = src_ref[b]
        hi = x.astype(jnp.bfloat16).astype(jnp.float32)
        mid = (x - hi).astype(jnp.bfloat16).astype(jnp.float32)
        lo = x - hi - mid
        out_ref[b] = (_dot_nt(g, hi) + _dot_nt(g, mid)) + _dot_nt(g, lo)


def kernel(src_pts, tgt_pts, W1, b1, W2, b2, Ws, bs):
    del tgt_pts, bs
    call = pl.pallas_call(
        _body,
        out_shape=jax.ShapeDtypeStruct((2, _K, 6), jnp.float32),
    )
    return call(src_pts, W1, b1[:, None], W2, b2[:, None], Ws)
